# baseline (speedup 1.0000x reference)
import jax
import jax.numpy as jnp
from jax import lax
from jax.experimental import pallas as pl
from jax.experimental.pallas import tpu as pltpu
from jax.experimental.pallas import tpu_sc as plsc

NC = 2
NS = 16
NW = NC * NS

B, L, E, K = 16384, 200, 4, 5
CH = 32
VECS_PER_ROW = L * E // 16
NSPLIT = 2
NB = B // NSPLIT


def _make_call(nb):
    rows_per_w = nb // NW
    nchunks = rows_per_w // CH

    def body(x_hbm, t_hbm, out_hbm, t_v, idx0, idx1, out0, out1,
             sin0, sin1, sout0, sout1):
        wid = lax.axis_index("s") * NC + lax.axis_index("c")
        pltpu.sync_copy(t_hbm, t_v)

        iota = lax.iota(jnp.int32, 16)
        quad = iota >> 2
        epat = iota & 3

        idxb, outb = (idx0, idx1), (out0, out1)
        sins, souts = (sin0, sin1), (sout0, sout1)

        def row0(c):
            return wid * rows_per_w + c * CH

        def compute(idx_v, out_v):
            def row_body(r, carry):
                rvec = jnp.full((16,), r, jnp.int32)

                @plsc.parallel_loop(0, VECS_PER_ROW, unroll=10)
                def vec_body(v):
                    colv = v * 4 + quad
                    base = plsc.load_gather(idx_v, [rvec, colv])
                    val = plsc.load_gather(t_v, [base + epat])
                    out_v[r, pl.ds(v * 16, 16)] = val

                return carry

            lax.fori_loop(0, CH, row_body, 0)

        pltpu.make_async_copy(x_hbm.at[pl.ds(row0(0), CH)], idx0, sin0).start()
        for c in range(nchunks):
            b = c & 1
            pltpu.make_async_copy(x_hbm.at[pl.ds(row0(c), CH)], idxb[b], sins[b]).wait()
            if c + 1 < nchunks:
                pltpu.make_async_copy(
                    x_hbm.at[pl.ds(row0(c + 1), CH)], idxb[1 - b], sins[1 - b]
                ).start()
            if c >= 2:
                pltpu.make_async_copy(
                    outb[b], out_hbm.at[pl.ds(row0(c - 2), CH)], souts[b]
                ).wait()
            compute(idxb[b], outb[b])
            pltpu.make_async_copy(outb[b], out_hbm.at[pl.ds(row0(c), CH)], souts[b]).start()

        for c in (nchunks - 2, nchunks - 1):
            b = c & 1
            pltpu.make_async_copy(outb[b], out_hbm.at[pl.ds(row0(c), CH)], souts[b]).wait()

    return pl.kernel(
        body,
        out_type=jax.ShapeDtypeStruct((nb, L * E), jnp.float32),
        mesh=plsc.VectorSubcoreMesh(core_axis_name="c", subcore_axis_name="s"),
        compiler_params=pltpu.CompilerParams(needs_layout_passes=False),
        scratch_types=[
            pltpu.VMEM((L * K * E,), jnp.float32),
            pltpu.VMEM((CH, L), jnp.int32),
            pltpu.VMEM((CH, L), jnp.int32),
            pltpu.VMEM((CH, L * E), jnp.float32),
            pltpu.VMEM((CH, L * E), jnp.float32),
            pltpu.SemaphoreType.DMA,
            pltpu.SemaphoreType.DMA,
            pltpu.SemaphoreType.DMA,
            pltpu.SemaphoreType.DMA,
        ],
    )


def kernel(X_nucl, We, position_encoding):
    x = X_nucl.astype(jnp.int32)
    xa = x * 4 + jnp.arange(L, dtype=jnp.int32)[None, :] * 20
    t = (position_encoding[0][:, None, :] + We[None, :, :]).reshape(-1)
    t = t.astype(jnp.float32)

    call = _make_call(NB)
    outs = [
        call(lax.slice_in_dim(xa, i * NB, (i + 1) * NB, axis=0), t).reshape(NB, L, E)
        for i in range(NSPLIT)
    ]
    return jnp.concatenate(outs, axis=0)


# speedup vs baseline: 1.0514x; 1.0514x over previous
import jax
import jax.numpy as jnp
from jax import lax
from jax.experimental import pallas as pl
from jax.experimental.pallas import tpu as pltpu
from jax.experimental.pallas import tpu_sc as plsc

NC = 2
NS = 16
NW = NC * NS

B, L, E, K = 16384, 200, 4, 5
ROWS_PER_W = B // NW
CH = 32
NCHUNKS = ROWS_PER_W // CH
VECS_PER_ROW = L * E // 16


def _sc_body(x_hbm, t_hbm, out_hbm, t_v, idx0, idx1, out0, out1,
             sin0, sin1, sout0, sout1, stab):
    wid = lax.axis_index("s") * NC + lax.axis_index("c")

    iota = lax.iota(jnp.int32, 16)
    quad = iota >> 2
    epat = iota & 3

    idxb, outb = (idx0, idx1), (out0, out1)
    sins, souts = (sin0, sin1), (sout0, sout1)

    def row0(c):
        return wid * ROWS_PER_W + c * CH

    def compute(idx_v, out_v):
        def row_body(r, carry):
            rvec = jnp.full((16,), r, jnp.int32)

            @plsc.parallel_loop(0, VECS_PER_ROW, unroll=10)
            def vec_body(v):
                colv = v * 4 + quad
                base = plsc.load_gather(idx_v, [rvec, colv])
                val = plsc.load_gather(t_v, [base + epat])
                out_v[r, pl.ds(v * 16, 16)] = val

            return carry

        lax.fori_loop(0, CH, row_body, 0)

    tcopy = pltpu.make_async_copy(t_hbm, t_v, stab)
    tcopy.start()
    pltpu.make_async_copy(x_hbm.at[pl.ds(row0(0), CH)], idx0, sin0).start()
    tcopy.wait()
    for c in range(NCHUNKS):
        b = c & 1
        pltpu.make_async_copy(x_hbm.at[pl.ds(row0(c), CH)], idxb[b], sins[b]).wait()
        if c + 1 < NCHUNKS:
            pltpu.make_async_copy(
                x_hbm.at[pl.ds(row0(c + 1), CH)], idxb[1 - b], sins[1 - b]
            ).start()
        if c >= 2:
            pltpu.make_async_copy(
                outb[b], out_hbm.at[pl.ds(row0(c - 2), CH)], souts[b]
            ).wait()
        compute(idxb[b], outb[b])
        pltpu.make_async_copy(outb[b], out_hbm.at[pl.ds(row0(c), CH)], souts[b]).start()

    for c in (NCHUNKS - 2, NCHUNKS - 1):
        b = c & 1
        pltpu.make_async_copy(outb[b], out_hbm.at[pl.ds(row0(c), CH)], souts[b]).wait()


def kernel(X_nucl, We, position_encoding):
    x = X_nucl.astype(jnp.int32)
    xa = x * 4 + jnp.arange(L, dtype=jnp.int32)[None, :] * 20
    t = (position_encoding[0][:, None, :] + We[None, :, :]).reshape(-1)
    t = t.astype(jnp.float32)

    call = pl.kernel(
        _sc_body,
        out_type=jax.ShapeDtypeStruct((B, L * E), jnp.float32),
        mesh=plsc.VectorSubcoreMesh(core_axis_name="c", subcore_axis_name="s"),
        compiler_params=pltpu.CompilerParams(needs_layout_passes=False),
        scratch_types=[
            pltpu.VMEM((L * K * E,), jnp.float32),
            pltpu.VMEM((CH, L), jnp.int32),
            pltpu.VMEM((CH, L), jnp.int32),
            pltpu.VMEM((CH, L * E), jnp.float32),
            pltpu.VMEM((CH, L * E), jnp.float32),
            pltpu.SemaphoreType.DMA,
            pltpu.SemaphoreType.DMA,
            pltpu.SemaphoreType.DMA,
            pltpu.SemaphoreType.DMA,
            pltpu.SemaphoreType.DMA,
        ],
    )
    out = call(xa, t)
    return out.reshape(B, L, E)


# speedup vs baseline: 1.0528x; 1.0014x over previous
import jax
import jax.numpy as jnp
from jax import lax
from jax.experimental import pallas as pl
from jax.experimental.pallas import tpu as pltpu
from jax.experimental.pallas import tpu_sc as plsc

NC = 2
NS = 16
NW = NC * NS

B, L, E, K = 16384, 200, 4, 5
ROWS_PER_W = B // NW
CH = 32
NCHUNKS = ROWS_PER_W // CH
VECS_PER_ROW = L * E // 16


def _sc_body(x_hbm, t_hbm, out_hbm, t_v, idx0, idx1, out0, out1,
             sin0, sin1, sout0, sout1):
    wid = lax.axis_index("s") * NC + lax.axis_index("c")

    iota = lax.iota(jnp.int32, 16)
    quad = iota >> 2
    epat = iota & 3

    idxb, outb = (idx0, idx1), (out0, out1)
    sins, souts = (sin0, sin1), (sout0, sout1)

    def row0(c):
        return wid * ROWS_PER_W + c * CH

    def compute(idx_v, out_v):
        def row_body(r, carry):
            rvec = jnp.full((16,), r, jnp.int32)

            @plsc.parallel_loop(0, VECS_PER_ROW, unroll=10)
            def vec_body(v):
                colv = v * 4 + quad
                base = plsc.load_gather(idx_v, [rvec, colv])
                val = plsc.load_gather(t_v, [base + epat])
                out_v[r, pl.ds(v * 16, 16)] = val

            return carry

        lax.fori_loop(0, CH, row_body, 0)

    pltpu.sync_copy(t_hbm, t_v)
    pltpu.make_async_copy(x_hbm.at[pl.ds(row0(0), CH)], idx0, sin0).start()
    for c in range(NCHUNKS):
        b = c & 1
        pltpu.make_async_copy(x_hbm.at[pl.ds(row0(c), CH)], idxb[b], sins[b]).wait()
        if c + 1 < NCHUNKS:
            pltpu.make_async_copy(
                x_hbm.at[pl.ds(row0(c + 1), CH)], idxb[1 - b], sins[1 - b]
            ).start()
        if c >= 2:
            pltpu.make_async_copy(
                outb[b], out_hbm.at[pl.ds(row0(c - 2), CH)], souts[b]
            ).wait()
        compute(idxb[b], outb[b])
        pltpu.make_async_copy(outb[b], out_hbm.at[pl.ds(row0(c), CH)], souts[b]).start()

    for c in (NCHUNKS - 2, NCHUNKS - 1):
        b = c & 1
        pltpu.make_async_copy(outb[b], out_hbm.at[pl.ds(row0(c), CH)], souts[b]).wait()


def kernel(X_nucl, We, position_encoding):
    x = X_nucl.astype(jnp.int32)
    xa = x * 4 + jnp.arange(L, dtype=jnp.int32)[None, :] * 20
    t = (position_encoding[0][:, None, :] + We[None, :, :]).reshape(-1)
    t = t.astype(jnp.float32)

    call = pl.kernel(
        _sc_body,
        out_type=jax.ShapeDtypeStruct((B, L * E), jnp.float32),
        mesh=plsc.VectorSubcoreMesh(core_axis_name="c", subcore_axis_name="s"),
        compiler_params=pltpu.CompilerParams(needs_layout_passes=False),
        scratch_types=[
            pltpu.VMEM((L * K * E,), jnp.float32),
            pltpu.VMEM((CH, L), jnp.int32),
            pltpu.VMEM((CH, L), jnp.int32),
            pltpu.VMEM((CH, L * E), jnp.float32),
            pltpu.VMEM((CH, L * E), jnp.float32),
            pltpu.SemaphoreType.DMA,
            pltpu.SemaphoreType.DMA,
            pltpu.SemaphoreType.DMA,
            pltpu.SemaphoreType.DMA,
        ],
    )
    out = call(xa, t)
    return out.reshape(B, L, E)


# speedup vs baseline: 1.0548x; 1.0019x over previous
import jax
import jax.numpy as jnp
from jax import lax
from jax.experimental import pallas as pl
from jax.experimental.pallas import tpu as pltpu
from jax.experimental.pallas import tpu_sc as plsc

NC = 2
NS = 16
NW = NC * NS

B, L, E, K = 16384, 200, 4, 5
ROWS_PER_W = B // NW
CH = 32
NCHUNKS = ROWS_PER_W // CH
VECS_PER_ROW = L * E // 16


def _sc_body(x_hbm, t_hbm, out_hbm, t_v, idx0, idx1, out0, out1,
             sin0, sin1, sout0, sout1, stab):
    wid = lax.axis_index("s") * NC + lax.axis_index("c")

    iota = lax.iota(jnp.int32, 16)
    quad = iota >> 2
    epat = iota & 3

    idxb, outb = (idx0, idx1), (out0, out1)
    sins, souts = (sin0, sin1), (sout0, sout1)

    def row0(c):
        return wid * ROWS_PER_W + c * CH

    def compute(idx_v, out_v):
        def row_body(r, carry):
            rvec = jnp.full((16,), r, jnp.int32)

            @plsc.parallel_loop(0, VECS_PER_ROW, unroll=10)
            def vec_body(v):
                colv = v * 4 + quad
                base = plsc.load_gather(idx_v, [rvec, colv])
                val = plsc.load_gather(t_v, [base + epat])
                out_v[r, pl.ds(v * 16, 16)] = val

            return carry

        lax.fori_loop(0, CH, row_body, 0)

    tcopy = pltpu.make_async_copy(t_hbm, t_v, stab)
    tcopy.start()
    pltpu.make_async_copy(x_hbm.at[pl.ds(row0(0), CH)], idx0, sin0).start()
    tcopy.wait()
    for c in range(NCHUNKS):
        b = c & 1
        pltpu.make_async_copy(x_hbm.at[pl.ds(row0(c), CH)], idxb[b], sins[b]).wait()
        if c + 1 < NCHUNKS:
            pltpu.make_async_copy(
                x_hbm.at[pl.ds(row0(c + 1), CH)], idxb[1 - b], sins[1 - b]
            ).start()
        if c >= 2:
            pltpu.make_async_copy(
                outb[b], out_hbm.at[pl.ds(row0(c - 2), CH)], souts[b]
            ).wait()
        compute(idxb[b], outb[b])
        pltpu.make_async_copy(outb[b], out_hbm.at[pl.ds(row0(c), CH)], souts[b]).start()

    for c in (NCHUNKS - 2, NCHUNKS - 1):
        b = c & 1
        pltpu.make_async_copy(outb[b], out_hbm.at[pl.ds(row0(c), CH)], souts[b]).wait()


def kernel(X_nucl, We, position_encoding):
    x = X_nucl.astype(jnp.int32)
    xa = x * 4 + jnp.arange(L, dtype=jnp.int32)[None, :] * 20
    t = (position_encoding[0][:, None, :] + We[None, :, :]).reshape(-1)
    t = t.astype(jnp.float32)

    call = pl.kernel(
        _sc_body,
        out_type=jax.ShapeDtypeStruct((B, L * E), jnp.float32),
        mesh=plsc.VectorSubcoreMesh(core_axis_name="c", subcore_axis_name="s"),
        compiler_params=pltpu.CompilerParams(needs_layout_passes=False),
        scratch_types=[
            pltpu.VMEM((L * K * E,), jnp.float32),
            pltpu.VMEM((CH, L), jnp.int32),
            pltpu.VMEM((CH, L), jnp.int32),
            pltpu.VMEM((CH, L * E), jnp.float32),
            pltpu.VMEM((CH, L * E), jnp.float32),
            pltpu.SemaphoreType.DMA,
            pltpu.SemaphoreType.DMA,
            pltpu.SemaphoreType.DMA,
            pltpu.SemaphoreType.DMA,
            pltpu.SemaphoreType.DMA,
        ],
    )
    out = call(xa, t)
    return out.reshape(B, L, E)
